# baseline (speedup 1.0000x reference)
import functools

import jax
import jax.numpy as jnp
from jax import lax
from jax.experimental import pallas as pl
from jax.experimental.pallas import tpu as pltpu
from jax.experimental.pallas import tpu_sc as plsc

VOCAB = 1000
D = 1000
BATCH = 1024
SEQ = 50
SEQP = 56
NC = 2
NS = 16
NW = NC * NS
BPW = BATCH // NW
NT = 7
ROWS = (8, 8, 8, 8, 8, 8, 2)

_mesh = plsc.VectorSubcoreMesh(
    core_axis_name="c", subcore_axis_name="s", num_cores=NC, num_subcores=NS
)

_scratch = (
    [pltpu.VMEM((BPW * SEQP,), jnp.int32)]
    + [pltpu.VMEM((8, 8, 128), jnp.float32) for _ in range(NT)]
    + [pltpu.VMEM((8, D), jnp.float32) for _ in range(NT)]
    + [pltpu.SemaphoreType.DMA for _ in range(2 * NT)]
)


@functools.partial(
    pl.kernel,
    mesh=_mesh,
    out_type=jax.ShapeDtypeStruct((BATCH, SEQ, D), jnp.float32),
    scratch_types=_scratch,
    compiler_params=pltpu.CompilerParams(needs_layout_passes=False),
)
def _emb_lookup(idx_hbm, table_hbm, out_hbm, idx_v, *rest):
    bufs = rest[0:NT]
    shfs = rest[NT : 2 * NT]
    gsems = rest[2 * NT : 3 * NT]
    ssems = rest[3 * NT : 4 * NT]

    wid = lax.axis_index("s") * NC + lax.axis_index("c")
    bbase = wid * BPW
    pltpu.sync_copy(idx_hbm.at[pl.ds(bbase * SEQP, BPW * SEQP)], idx_v)

    def gdesc(b, t):
        n = ROWS[t]
        dst = bufs[t] if n == 8 else bufs[t].at[pl.ds(0, n)]
        return pltpu.make_async_copy(
            table_hbm.at[idx_v.at[pl.ds(b * SEQP + 8 * t, n)]], dst, gsems[t]
        )

    def sdesc(b, t):
        n = ROWS[t]
        src = shfs[t] if n == 8 else shfs[t].at[pl.ds(0, n)]
        return pltpu.make_async_copy(
            src, out_hbm.at[bbase + b, pl.ds(8 * t, n)], ssems[t]
        )

    def shuffle(t):
        def row(r, carry):
            for j in range(7):
                for k in range(8):
                    c = j * 128 + k * 16
                    shfs[t][r, pl.ds(c, 16)] = bufs[t][r, j, pl.ds(k * 16, 16)]
            for k in range(6):
                c = 896 + k * 16
                shfs[t][r, pl.ds(c, 16)] = bufs[t][r, 7, pl.ds(k * 16, 16)]
            lanes = lax.iota(jnp.int32, 16)
            msk = lanes < 8
            cols = jnp.where(msk, 992 + lanes, 999)
            rows = jnp.full((16,), r, jnp.int32)
            vals = bufs[t][r, 7, pl.ds(96, 16)]
            plsc.store_scatter(shfs[t], [rows, cols], vals, mask=msk)
            return carry

        lax.fori_loop(0, ROWS[t], row, 0)

    for t in range(NT):
        gdesc(0, t).start()
    for t in range(NT):
        gdesc(0, t).wait()
        shuffle(t)
        sdesc(0, t).start()

    def batch(b, carry):
        for t in range(NT):
            gdesc(b, t).start()
        for t in range(NT):
            gdesc(b, t).wait()
            sdesc(b - 1, t).wait()
            shuffle(t)
            sdesc(b, t).start()
        return carry

    lax.fori_loop(1, BPW, batch, 0)

    for t in range(NT):
        sdesc(BPW - 1, t).wait()


def kernel(x, embedding):
    idxp = jnp.pad(x.astype(jnp.int32), ((0, 0), (0, SEQP - SEQ))).reshape(-1)
    tab3 = jnp.pad(embedding, ((0, 0), (0, 1024 - D))).reshape(VOCAB, 8, 128)
    return _emb_lookup(idxp, tab3)
